# baseline (speedup 1.0000x reference)
import functools
import math

import jax
import jax.numpy as jnp
from jax import lax
from jax.experimental import pallas as pl
from jax.experimental.pallas import tpu as pltpu
from jax.experimental.pallas import tpu_sc as plsc

N = 10000
E = 320000
D = 128
NUM_GRAPHS = 8

NTILES = 16
NSC = 2
B = 128
PER_TILE = E // NTILES
GB = 8
TPB = GB * ((PER_TILE + B * GB - 1) // (B * GB))
NGRP = TPB // GB
PT_PAD = TPB * B
N_PAD = 10240
ROWS_PER_TILE = N_PAD // NTILES


def _pad_core_tile_idx(idx, fill, ncore_splits):
    if ncore_splits == 1:
        per_tile = E // NTILES
        chunks = jnp.broadcast_to(idx.reshape(1, NTILES, per_tile), (2, NTILES, per_tile))
    else:
        per_tile = E // (2 * NTILES)
        chunks = idx.reshape(2, NTILES, per_tile)
    tpb = GB * ((per_tile + B * GB - 1) // (B * GB))
    pad = jnp.full((2, NTILES, tpb * B - per_tile), fill, jnp.int32)
    return jnp.concatenate([chunks, pad], axis=2).reshape(2, NTILES, tpb, B)


def _make_sc_scatter(tpb):
    mesh = plsc.VectorSubcoreMesh(core_axis_name="c", subcore_axis_name="s")
    ngrp = tpb // GB

    @functools.partial(
        pl.kernel,
        out_type=jax.ShapeDtypeStruct((NSC, N_PAD, 128), jnp.float32),
        mesh=mesh,
        scratch_types=[
            pltpu.VMEM((GB, B), jnp.int32),
            pltpu.VMEM((GB, B), jnp.int32),
            pltpu.VMEM((B, 128), jnp.float32),
            pltpu.VMEM_SHARED((N_PAD, 128), jnp.float32),
            pltpu.SemaphoreType.DMA,
        ],
    )
    def body(y_hbm, srcs_hbm, dsts_hbm, zeros_hbm, out_hbm, src_v, dst_v, rows_v, acc_sh, sem):
        c = lax.axis_index("c")
        s = lax.axis_index("s")
        r0 = s * ROWS_PER_TILE
        pltpu.sync_copy(zeros_hbm.at[pl.ds(r0, ROWS_PER_TILE)],
                        acc_sh.at[pl.ds(r0, ROWS_PER_TILE)])
        plsc.subcore_barrier()

        def group(g, carry):
            pltpu.sync_copy(srcs_hbm.at[c, s, pl.ds(g * GB, GB)], src_v)
            pltpu.sync_copy(dsts_hbm.at[c, s, pl.ds(g * GB, GB)], dst_v)

            def step(b, carry2):
                pltpu.async_copy(y_hbm.at[src_v.at[b]], rows_v, sem).wait()
                pltpu.sync_copy(rows_v, acc_sh.at[dst_v.at[b]], add=True)
                return carry2

            return lax.fori_loop(0, GB, step, carry, unroll=False)

        lax.fori_loop(0, ngrp, group, 0, unroll=False)
        plsc.subcore_barrier()
        pltpu.sync_copy(acc_sh.at[pl.ds(r0, ROWS_PER_TILE)],
                        out_hbm.at[c, pl.ds(r0, ROWS_PER_TILE)])

    return body


_make_sc_scatter = functools.lru_cache(None)(_make_sc_scatter)

TPB1 = 160
NGRP1 = TPB1 // GB
EW_PAD = (NTILES - 1) * PER_TILE + TPB1 * B


def _sc_l1_scatter():
    mesh = plsc.VectorSubcoreMesh(core_axis_name="c", subcore_axis_name="s")

    @functools.partial(
        pl.kernel,
        out_type=jax.ShapeDtypeStruct((4, N_PAD, 128), jnp.float32),
        mesh=mesh,
        scratch_types=[
            pltpu.VMEM((GB, B), jnp.int32),
            pltpu.VMEM((GB, B), jnp.int32),
            pltpu.VMEM((GB * B,), jnp.float32),
            pltpu.VMEM((B, 128), jnp.float32),
            pltpu.VMEM_SHARED((N_PAD, 128), jnp.float32),
            pltpu.SemaphoreType.DMA,
        ],
    )
    def body(xs_hbm, srcs_hbm, dsts_hbm, ewt_hbm, zeros_hbm, out_hbm,
             src_v, dst_v, ew_v, rows_v, acc_sh, sem):
        c = lax.axis_index("c")
        s = lax.axis_index("s")
        r0 = s * ROWS_PER_TILE

        for r in range(2):
            k = 2 * c + r
            pltpu.sync_copy(zeros_hbm.at[pl.ds(r0, ROWS_PER_TILE)],
                            acc_sh.at[pl.ds(r0, ROWS_PER_TILE)])
            plsc.subcore_barrier()

            def group(g, carry):
                pltpu.sync_copy(srcs_hbm.at[k, s, pl.ds(g * GB, GB)], src_v)
                pltpu.sync_copy(dsts_hbm.at[c, s, pl.ds(g * GB, GB)], dst_v)
                pltpu.sync_copy(ewt_hbm.at[k, s, pl.ds(g * (GB * B), GB * B)], ew_v)

                def step2(b, carry2):
                    pltpu.async_copy(xs_hbm.at[src_v.at[b]], rows_v, sem).wait()

                    def scale(t, carry3):
                        w16 = ew_v[pl.ds(b * B + t * 16, 16)]
                        for lane in range(16):
                            w = w16[lane]
                            j = t * 16 + lane
                            for h in range(8):
                                rows_v[j, pl.ds(h * 16, 16)] = rows_v[j, pl.ds(h * 16, 16)] * w
                        return carry3

                    lax.fori_loop(0, B // 16, scale, 0, unroll=False)
                    pltpu.sync_copy(rows_v, acc_sh.at[dst_v.at[b]], add=True)
                    return carry2

                return lax.fori_loop(0, GB, step2, carry, unroll=False)

            lax.fori_loop(0, NGRP1, group, 0, unroll=False)
            plsc.subcore_barrier()
            pltpu.sync_copy(acc_sh.at[pl.ds(r0, ROWS_PER_TILE)],
                            out_hbm.at[k, pl.ds(r0, ROWS_PER_TILE)])
            plsc.subcore_barrier()

    return body


_sc_l1_scatter = functools.lru_cache(None)(_sc_l1_scatter)


def _gcn_scatter(vals, dst, n):
    return jnp.zeros((n, vals.shape[1]), vals.dtype).at[dst].add(vals)


N2 = 10240
RB = 1000
NBLK = N // RB
INT_MIN = -2147483648


def _monokey(s):
    b = lax.bitcast_convert_type(s, jnp.int32)
    return jnp.where(b >= 0, b, jnp.int32(INT_MIN) - b)


def _bisect(pred_count, k, lo0, hi0, iters):
    def it(_, lohi):
        lo, hi = lohi
        mid = (lo >> 1) + (hi >> 1) + (lo & hi & 1)
        cnt = pred_count(mid)
        big = cnt >= k
        return jnp.where(big, mid + 1, lo), jnp.where(big, hi, mid)

    lo, hi = lax.fori_loop(0, iters, it, (lo0, hi0))
    return lo


def _select_kernel(pres_ref, batch_ref, s_ref, w_ref, m_ref):
    score = jnp.tanh(pres_ref[...])
    key = _monokey(score)
    batch = batch_ref[...]
    idx2d = (lax.broadcasted_iota(jnp.int32, (80, 128), 0) * 128
             + lax.broadcasted_iota(jnp.int32, (80, 128), 1))
    sel = jnp.zeros((80, 128), jnp.bool_)
    for g in range(NUM_GRAPHS):
        ing = batch == g
        cnt_g = jnp.sum(ing.astype(jnp.float32))
        k_g = jnp.floor((cnt_g + 1.0) * 0.5)
        kslab = jnp.where(ing, key, jnp.int32(INT_MIN))

        def cnt_gt(t):
            return jnp.sum((kslab > t).astype(jnp.float32))

        v_g = _bisect(cnt_gt, k_g, jnp.int32(INT_MIN), jnp.int32(2147483647), 32)
        j_g = k_g - cnt_gt(v_g)

        def cnt_tie(t):
            return jnp.sum(((kslab == v_g) & (idx2d < t)).astype(jnp.float32))

        c_g = _bisect(lambda t: j_g - cnt_tie(t), jnp.float32(1.0),
                      jnp.int32(0), jnp.int32(N2), 15)
        sel = sel | (ing & ((key > v_g) | ((key == v_g) & (idx2d < c_g))))
    s_ref[...] = score
    w_ref[...] = jnp.where(sel, score, 0.0)
    m_ref[...] = jnp.where(sel, 0.0, -jnp.inf)


def _readout_kernel(oh_ref, scol_ref, wcol_ref, mcol_ref,
                    g1a_ref, g1b_ref, g1c_ref, g1d_ref, g2a_ref, g2b_ref, g3_ref,
                    wm1_ref, bm1_ref, wm2_ref, bm2_ref, wm3_ref, bm3_ref,
                    o_ref, mean_acc, max_acc, kvec):
    i = pl.program_id(0)

    @pl.when(i == 0)
    def _init():
        mean_acc[...] = jnp.zeros((NUM_GRAPHS, 7 * D), jnp.float32)
        max_acc[...] = jnp.full((NUM_GRAPHS, 7 * D), -jnp.inf, jnp.float32)
        kvec[...] = jnp.zeros((1, NUM_GRAPHS), jnp.float32)

    oh_blk = oh_ref[...]
    gcn_blk = jnp.concatenate(
        [g1a_ref[...], g1b_ref[...], g1c_ref[...], g1d_ref[...],
         g2a_ref[...], g2b_ref[...], g3_ref[...]], axis=1)
    sw_blk = oh_blk * wcol_ref[...]
    mean_acc[...] += lax.dot_general(sw_blk, gcn_blk, (((0,), (0,)), ((), ())))
    selcol = jnp.where(mcol_ref[...] == 0.0, 1.0, 0.0)
    kvec[...] += lax.dot_general(selcol, oh_blk, (((0,), (0,)), ((), ())))
    cand = gcn_blk * scol_ref[...] + mcol_ref[...]
    for g in range(NUM_GRAPHS):
        ing = oh_blk[:, g:g + 1] > 0.0
        red = jnp.max(jnp.where(ing, cand, -jnp.inf), axis=0, keepdims=True)
        max_acc[pl.ds(g, 1)] = jnp.maximum(max_acc[pl.ds(g, 1)], red)

    @pl.when(i == NBLK - 1)
    def _mlp():
        kc = kvec[...].reshape(NUM_GRAPHS, 1)
        mean = mean_acc[...] / jnp.maximum(kc, 1.0)
        readout = jnp.concatenate([mean, max_acc[...]], axis=1)
        h = jax.nn.relu(readout @ wm1_ref[...] + bm1_ref[...])
        h = jax.nn.relu(h @ wm2_ref[...] + bm2_ref[...])
        o_ref[...] = h @ wm3_ref[...] + bm3_ref[...]


def _pool_readout(pres, batch, pieces, Wm1, bm1, Wm2, bm2, Wm3, bm3):
    n = pres.shape[0]
    padn = N2 - n
    pres2d = jnp.pad(pres, (0, padn)).reshape(80, 128)
    batch2d = jnp.pad(batch, (0, padn), constant_values=NUM_GRAPHS).reshape(80, 128)
    oh = (batch[:, None] == jnp.arange(NUM_GRAPHS)[None]).astype(jnp.float32)

    s2d, w2d, m2d = pl.pallas_call(
        _select_kernel,
        out_shape=[jax.ShapeDtypeStruct((80, 128), jnp.float32)] * 3,
    )(pres2d, batch2d)
    scol = s2d.reshape(N2, 1)[:n]
    wcol = w2d.reshape(N2, 1)[:n]
    mcol = m2d.reshape(N2, 1)[:n]

    whole = lambda shape: pl.BlockSpec(shape, lambda i: tuple(0 for _ in shape))
    blk = lambda w: pl.BlockSpec((RB, w), lambda i: (i, 0))
    return pl.pallas_call(
        _readout_kernel,
        grid=(NBLK,),
        in_specs=[
            blk(NUM_GRAPHS), blk(1), blk(1), blk(1),
            blk(D), blk(D), blk(D), blk(D), blk(D), blk(D), blk(D),
            whole((14 * D, 32)), whole((1, 32)), whole((32, 8)), whole((1, 8)),
            whole((8, 2)), whole((1, 2)),
        ],
        out_specs=pl.BlockSpec((NUM_GRAPHS, 2), lambda i: (0, 0)),
        out_shape=jax.ShapeDtypeStruct((NUM_GRAPHS, 2), jnp.float32),
        scratch_shapes=[
            pltpu.VMEM((NUM_GRAPHS, 7 * D), jnp.float32),
            pltpu.VMEM((NUM_GRAPHS, 7 * D), jnp.float32),
            pltpu.VMEM((1, NUM_GRAPHS), jnp.float32),
        ],
    )(oh, scol, wcol, mcol, *pieces,
      Wm1, bm1.reshape(1, -1), Wm2, bm2.reshape(1, -1), Wm3, bm3.reshape(1, -1))


def _mlp_kernel(r_ref, wm1_ref, bm1_ref, wm2_ref, bm2_ref, wm3_ref, bm3_ref, o_ref):
    h = jax.nn.relu(r_ref[...] @ wm1_ref[...] + bm1_ref[...])
    h = jax.nn.relu(h @ wm2_ref[...] + bm2_ref[...])
    o_ref[...] = h @ wm3_ref[...] + bm3_ref[...]


def _col_halves(y, width):
    n = y.shape[0]
    return y.reshape(n, 2, width).transpose(1, 0, 2).reshape(2 * n, width)


def kernel(x, edge_index, edge_attr, batch, W1A, b1A, W1B, b1B, W1C, b1C, W1D, b1D, W2, b2, W3, b3, Wp_rel, bp_rel, Wp_root, Wm1, bm1, Wm2, bm2, Wm3, bm3):
    n = x.shape[0]
    src = edge_index[0]
    dst = edge_index[1]

    counts = jnp.bincount(batch, length=NUM_GRAPHS)
    starts = jnp.cumsum(counts) - counts

    srcs_all = _pad_core_tile_idx(src, 0, 1)
    srcs_all = srcs_all.at[1].add(n)
    dsts_all = _pad_core_tile_idx(dst, N, 1)
    srcs_half = _pad_core_tile_idx(src, 0, 2)
    dsts_half = _pad_core_tile_idx(dst, N, 2)
    srcs_l1 = (_pad_core_tile_idx(src, 0, 1)[0][None]
               + (jnp.arange(4, dtype=jnp.int32) * n)[:, None, None, None])
    ew_chunks = edge_attr.T.reshape(4, NTILES, PER_TILE)
    ewt = jnp.concatenate(
        [ew_chunks, jnp.zeros((4, NTILES, TPB1 * B - PER_TILE), jnp.float32)],
        axis=2)
    zeros128 = jnp.zeros((N_PAD, 128), jnp.float32)

    ones_e = jnp.ones((E, 1), jnp.float32)
    vals5 = jnp.concatenate([edge_attr, ones_e], axis=1)
    deg5 = jnp.zeros((n, 5), jnp.float32).at[dst].add(vals5) + 1.0
    dinv5 = lax.rsqrt(deg5)

    W1k = jnp.stack([W1A, W1B, W1C, W1D])
    b1k = jnp.stack([b1A, b1B, b1C, b1D])[:, None]
    dinv4k = dinv5.T[:4, :, None]
    y1k = jnp.einsum('nd,kdw->knw', x, W1k) * dinv4k
    acc1 = _sc_l1_scatter()(y1k.reshape(4 * n, D), srcs_l1, dsts_all, ewt,
                            zeros128)
    gcn1k = jax.nn.relu((acc1[:, :n] + y1k) * dinv4k + b1k)

    dinv_o = dinv5[:, 4:5]
    y2h = jnp.einsum('knd,kdhw->hnw', gcn1k, W2.reshape(4, D, 2, D)) \
        * dinv_o[None]
    acc2 = _make_sc_scatter(160)(y2h.reshape(2 * n, D), srcs_all, dsts_all, zeros128)
    b2h = b2.reshape(2, 1, D)
    gcn2h = jax.nn.relu((acc2[:, :n] + y2h) * dinv_o[None] + b2h)

    y3 = jnp.einsum('hnd,hdw->nw', gcn2h, W3.reshape(2, D, D)) * dinv_o
    acc3 = _make_sc_scatter(80)(y3, srcs_half, dsts_half, zeros128)
    gcn3 = (acc3[0, :n] + acc3[1, :n] + y3) * dinv_o + b3

    def _piece_dot(wvec):
        w1 = wvec[:4 * D].reshape(4, D)
        w2 = wvec[4 * D:6 * D].reshape(2, D)
        return (jnp.einsum('knd,kd->n', gcn1k, w1)
                + jnp.einsum('hnd,hd->n', gcn2h, w2)
                + gcn3 @ wvec[6 * D:])

    s_rel = _piece_dot(Wp_rel[:, 0])
    s_root = _piece_dot(Wp_root[:, 0])
    agg_s = jnp.zeros((n,), jnp.float32).at[dst].add(s_rel[src])
    pres = agg_s + bp_rel[0] + s_root

    pieces = (gcn1k[0], gcn1k[1], gcn1k[2], gcn1k[3], gcn2h[0], gcn2h[1], gcn3)
    return _pool_readout(pres, batch, pieces, Wm1, bm1, Wm2, bm2, Wm3, bm3)


# speedup vs baseline: 1.3392x; 1.3392x over previous
import functools
import math

import jax
import jax.numpy as jnp
from jax import lax
from jax.experimental import pallas as pl
from jax.experimental.pallas import tpu as pltpu
from jax.experimental.pallas import tpu_sc as plsc

N = 10000
E = 320000
D = 128
NUM_GRAPHS = 8

NTILES = 16
NSC = 2
B = 128
PER_TILE = E // NTILES
GB = 8
TPB = GB * ((PER_TILE + B * GB - 1) // (B * GB))
NGRP = TPB // GB
PT_PAD = TPB * B
N_PAD = 10240
ROWS_PER_TILE = N_PAD // NTILES


def _pad_core_tile_idx(idx, fill, ncore_splits):
    if ncore_splits == 1:
        per_tile = E // NTILES
        chunks = jnp.broadcast_to(idx.reshape(1, NTILES, per_tile), (2, NTILES, per_tile))
    else:
        per_tile = E // (2 * NTILES)
        chunks = idx.reshape(2, NTILES, per_tile)
    tpb = GB * ((per_tile + B * GB - 1) // (B * GB))
    pad = jnp.full((2, NTILES, tpb * B - per_tile), fill, jnp.int32)
    return jnp.concatenate([chunks, pad], axis=2).reshape(2, NTILES, tpb, B)


def _make_sc_scatter(tpb):
    mesh = plsc.VectorSubcoreMesh(core_axis_name="c", subcore_axis_name="s")
    ngrp = tpb // GB

    @functools.partial(
        pl.kernel,
        out_type=jax.ShapeDtypeStruct((NSC, N_PAD, 128), jnp.float32),
        mesh=mesh,
        scratch_types=[
            pltpu.VMEM((GB, B), jnp.int32),
            pltpu.VMEM((GB, B), jnp.int32),
            pltpu.VMEM((B, 128), jnp.float32),
            pltpu.VMEM_SHARED((N_PAD, 128), jnp.float32),
            pltpu.SemaphoreType.DMA,
        ],
    )
    def body(y_hbm, srcs_hbm, dsts_hbm, zeros_hbm, out_hbm, src_v, dst_v, rows_v, acc_sh, sem):
        c = lax.axis_index("c")
        s = lax.axis_index("s")
        r0 = s * ROWS_PER_TILE
        pltpu.sync_copy(zeros_hbm.at[pl.ds(r0, ROWS_PER_TILE)],
                        acc_sh.at[pl.ds(r0, ROWS_PER_TILE)])
        plsc.subcore_barrier()

        def group(g, carry):
            pltpu.sync_copy(srcs_hbm.at[c, s, pl.ds(g * GB, GB)], src_v)
            pltpu.sync_copy(dsts_hbm.at[c, s, pl.ds(g * GB, GB)], dst_v)

            def step(b, carry2):
                pltpu.async_copy(y_hbm.at[src_v.at[b]], rows_v, sem).wait()
                pltpu.sync_copy(rows_v, acc_sh.at[dst_v.at[b]], add=True)
                return carry2

            return lax.fori_loop(0, GB, step, carry, unroll=False)

        lax.fori_loop(0, ngrp, group, 0, unroll=False)
        plsc.subcore_barrier()
        pltpu.sync_copy(acc_sh.at[pl.ds(r0, ROWS_PER_TILE)],
                        out_hbm.at[c, pl.ds(r0, ROWS_PER_TILE)])

    return body


_make_sc_scatter = functools.lru_cache(None)(_make_sc_scatter)

TPB1 = 160
NGRP1 = TPB1 // GB
EW_PAD = (NTILES - 1) * PER_TILE + TPB1 * B


def _sc_l1_scatter():
    mesh = plsc.VectorSubcoreMesh(core_axis_name="c", subcore_axis_name="s")

    @functools.partial(
        pl.kernel,
        out_type=jax.ShapeDtypeStruct((4, N_PAD, 128), jnp.float32),
        mesh=mesh,
        scratch_types=[
            pltpu.VMEM((GB, B), jnp.int32),
            pltpu.VMEM((GB, B), jnp.int32),
            pltpu.VMEM((GB * B,), jnp.float32),
            pltpu.VMEM((B, 128), jnp.float32),
            pltpu.VMEM_SHARED((N_PAD, 128), jnp.float32),
            pltpu.SemaphoreType.DMA,
        ],
    )
    def body(xs_hbm, srcs_hbm, dsts_hbm, ewt_hbm, zeros_hbm, out_hbm,
             src_v, dst_v, ew_v, rows_v, acc_sh, sem):
        c = lax.axis_index("c")
        s = lax.axis_index("s")
        r0 = s * ROWS_PER_TILE

        for r in range(2):
            k = 2 * c + r
            pltpu.sync_copy(zeros_hbm.at[pl.ds(r0, ROWS_PER_TILE)],
                            acc_sh.at[pl.ds(r0, ROWS_PER_TILE)])
            plsc.subcore_barrier()

            def group(g, carry):
                pltpu.sync_copy(srcs_hbm.at[k, s, pl.ds(g * GB, GB)], src_v)
                pltpu.sync_copy(dsts_hbm.at[c, s, pl.ds(g * GB, GB)], dst_v)
                pltpu.sync_copy(ewt_hbm.at[k, s, pl.ds(g * (GB * B), GB * B)], ew_v)

                def step2(b, carry2):
                    pltpu.async_copy(xs_hbm.at[src_v.at[b]], rows_v, sem).wait()

                    def scale(t, carry3):
                        w16 = ew_v[pl.ds(b * B + t * 16, 16)]
                        for lane in range(16):
                            w = w16[lane]
                            j = t * 16 + lane
                            for h in range(8):
                                rows_v[j, pl.ds(h * 16, 16)] = rows_v[j, pl.ds(h * 16, 16)] * w
                        return carry3

                    lax.fori_loop(0, B // 16, scale, 0, unroll=False)
                    pltpu.sync_copy(rows_v, acc_sh.at[dst_v.at[b]], add=True)
                    return carry2

                return lax.fori_loop(0, GB, step2, carry, unroll=False)

            lax.fori_loop(0, NGRP1, group, 0, unroll=False)
            plsc.subcore_barrier()
            pltpu.sync_copy(acc_sh.at[pl.ds(r0, ROWS_PER_TILE)],
                            out_hbm.at[k, pl.ds(r0, ROWS_PER_TILE)])
            plsc.subcore_barrier()

    return body


_sc_l1_scatter = functools.lru_cache(None)(_sc_l1_scatter)


def _gcn_scatter(vals, dst, n):
    return jnp.zeros((n, vals.shape[1]), vals.dtype).at[dst].add(vals)


N2 = 10240
RB = 1000
NBLK = N // RB
INT_MIN = -2147483648


def _monokey(s):
    b = lax.bitcast_convert_type(s, jnp.int32)
    return jnp.where(b >= 0, b, jnp.int32(INT_MIN) - b)


def _bisect(pred_count, k, lo0, hi0, iters):
    def it(_, lohi):
        lo, hi = lohi
        mid = (lo >> 1) + (hi >> 1) + (lo & hi & 1)
        cnt = pred_count(mid)
        big = cnt >= k
        return jnp.where(big, mid + 1, lo), jnp.where(big, hi, mid)

    lo, hi = lax.fori_loop(0, iters, it, (lo0, hi0))
    return lo


def _select_kernel(pres_ref, batch_ref, s_ref, w_ref, m_ref):
    score = jnp.tanh(pres_ref[...])
    key = _monokey(score)
    batch = batch_ref[...]
    idx2d = (lax.broadcasted_iota(jnp.int32, (80, 128), 0) * 128
             + lax.broadcasted_iota(jnp.int32, (80, 128), 1))
    sel = jnp.zeros((80, 128), jnp.bool_)
    for g in range(NUM_GRAPHS):
        ing = batch == g
        cnt_g = jnp.sum(ing.astype(jnp.float32))
        k_g = jnp.floor((cnt_g + 1.0) * 0.5)
        kslab = jnp.where(ing, key, jnp.int32(INT_MIN))

        def cnt_gt(t):
            return jnp.sum((kslab > t).astype(jnp.float32))

        v_g = _bisect(cnt_gt, k_g, jnp.int32(INT_MIN), jnp.int32(2147483647), 32)
        j_g = k_g - cnt_gt(v_g)

        def cnt_tie(t):
            return jnp.sum(((kslab == v_g) & (idx2d < t)).astype(jnp.float32))

        c_g = _bisect(lambda t: j_g - cnt_tie(t), jnp.float32(1.0),
                      jnp.int32(0), jnp.int32(N2), 15)
        sel = sel | (ing & ((key > v_g) | ((key == v_g) & (idx2d < c_g))))
    s_ref[...] = score
    w_ref[...] = jnp.where(sel, score, 0.0)
    m_ref[...] = jnp.where(sel, 0.0, -jnp.inf)


def _readout_kernel(oh_ref, scol_ref, wcol_ref, mcol_ref,
                    g1a_ref, g1b_ref, g1c_ref, g1d_ref, g2a_ref, g2b_ref, g3_ref,
                    wm1_ref, bm1_ref, wm2_ref, bm2_ref, wm3_ref, bm3_ref,
                    o_ref, mean_acc, max_acc, kvec):
    i = pl.program_id(0)

    @pl.when(i == 0)
    def _init():
        mean_acc[...] = jnp.zeros((NUM_GRAPHS, 7 * D), jnp.float32)
        max_acc[...] = jnp.full((NUM_GRAPHS, 7 * D), -jnp.inf, jnp.float32)
        kvec[...] = jnp.zeros((1, NUM_GRAPHS), jnp.float32)

    oh_blk = oh_ref[...]
    gcn_blk = jnp.concatenate(
        [g1a_ref[...], g1b_ref[...], g1c_ref[...], g1d_ref[...],
         g2a_ref[...], g2b_ref[...], g3_ref[...]], axis=1)
    sw_blk = oh_blk * wcol_ref[...]
    mean_acc[...] += lax.dot_general(sw_blk, gcn_blk, (((0,), (0,)), ((), ())))
    selcol = jnp.where(mcol_ref[...] == 0.0, 1.0, 0.0)
    kvec[...] += lax.dot_general(selcol, oh_blk, (((0,), (0,)), ((), ())))
    cand = gcn_blk * scol_ref[...] + mcol_ref[...]
    for g in range(NUM_GRAPHS):
        ing = oh_blk[:, g:g + 1] > 0.0
        red = jnp.max(jnp.where(ing, cand, -jnp.inf), axis=0, keepdims=True)
        max_acc[pl.ds(g, 1)] = jnp.maximum(max_acc[pl.ds(g, 1)], red)

    @pl.when(i == NBLK - 1)
    def _mlp():
        kc = kvec[...].reshape(NUM_GRAPHS, 1)
        mean = mean_acc[...] / jnp.maximum(kc, 1.0)
        readout = jnp.concatenate([mean, max_acc[...]], axis=1)
        h = jax.nn.relu(readout @ wm1_ref[...] + bm1_ref[...])
        h = jax.nn.relu(h @ wm2_ref[...] + bm2_ref[...])
        o_ref[...] = h @ wm3_ref[...] + bm3_ref[...]


def _pool_readout(pres, batch, pieces, Wm1, bm1, Wm2, bm2, Wm3, bm3):
    n = pres.shape[0]
    padn = N2 - n
    pres2d = jnp.pad(pres, (0, padn)).reshape(80, 128)
    batch2d = jnp.pad(batch, (0, padn), constant_values=NUM_GRAPHS).reshape(80, 128)
    oh = (batch[:, None] == jnp.arange(NUM_GRAPHS)[None]).astype(jnp.float32)

    s2d, w2d, m2d = pl.pallas_call(
        _select_kernel,
        out_shape=[jax.ShapeDtypeStruct((80, 128), jnp.float32)] * 3,
    )(pres2d, batch2d)
    scol = s2d.reshape(N2, 1)[:n]
    wcol = w2d.reshape(N2, 1)[:n]
    mcol = m2d.reshape(N2, 1)[:n]

    whole = lambda shape: pl.BlockSpec(shape, lambda i: tuple(0 for _ in shape))
    blk = lambda w: pl.BlockSpec((RB, w), lambda i: (i, 0))
    return pl.pallas_call(
        _readout_kernel,
        grid=(NBLK,),
        in_specs=[
            blk(NUM_GRAPHS), blk(1), blk(1), blk(1),
            blk(D), blk(D), blk(D), blk(D), blk(D), blk(D), blk(D),
            whole((14 * D, 32)), whole((1, 32)), whole((32, 8)), whole((1, 8)),
            whole((8, 2)), whole((1, 2)),
        ],
        out_specs=pl.BlockSpec((NUM_GRAPHS, 2), lambda i: (0, 0)),
        out_shape=jax.ShapeDtypeStruct((NUM_GRAPHS, 2), jnp.float32),
        scratch_shapes=[
            pltpu.VMEM((NUM_GRAPHS, 7 * D), jnp.float32),
            pltpu.VMEM((NUM_GRAPHS, 7 * D), jnp.float32),
            pltpu.VMEM((1, NUM_GRAPHS), jnp.float32),
        ],
    )(oh, scol, wcol, mcol, *pieces,
      Wm1, bm1.reshape(1, -1), Wm2, bm2.reshape(1, -1), Wm3, bm3.reshape(1, -1))


def _mlp_kernel(r_ref, wm1_ref, bm1_ref, wm2_ref, bm2_ref, wm3_ref, bm3_ref, o_ref):
    h = jax.nn.relu(r_ref[...] @ wm1_ref[...] + bm1_ref[...])
    h = jax.nn.relu(h @ wm2_ref[...] + bm2_ref[...])
    o_ref[...] = h @ wm3_ref[...] + bm3_ref[...]


def _col_halves(y, width):
    n = y.shape[0]
    return y.reshape(n, 2, width).transpose(1, 0, 2).reshape(2 * n, width)


def kernel(x, edge_index, edge_attr, batch, W1A, b1A, W1B, b1B, W1C, b1C, W1D, b1D, W2, b2, W3, b3, Wp_rel, bp_rel, Wp_root, Wm1, bm1, Wm2, bm2, Wm3, bm3):
    n = x.shape[0]
    src = edge_index[0]
    dst = edge_index[1]

    counts = jnp.bincount(batch, length=NUM_GRAPHS)
    starts = jnp.cumsum(counts) - counts

    srcs_all = _pad_core_tile_idx(src, 0, 1)
    srcs_all = srcs_all.at[1].add(n)
    dsts_all = _pad_core_tile_idx(dst, N, 1)
    srcs_half = _pad_core_tile_idx(src, 0, 2)
    dsts_half = _pad_core_tile_idx(dst, N, 2)
    srcs_l1 = (_pad_core_tile_idx(src, 0, 1)[0][None]
               + (jnp.arange(4, dtype=jnp.int32) * n)[:, None, None, None])
    ew_chunks = edge_attr.T.reshape(4, NTILES, PER_TILE)
    ewt = jnp.concatenate(
        [ew_chunks, jnp.zeros((4, NTILES, TPB1 * B - PER_TILE), jnp.float32)],
        axis=2)
    zeros128 = jnp.zeros((N_PAD, 128), jnp.float32)

    ones_e = jnp.ones((E, 1), jnp.float32)
    vals5 = jnp.concatenate([edge_attr, ones_e], axis=1)
    deg5 = jnp.zeros((n, 5), jnp.float32).at[dst].add(vals5) + 1.0
    dinv5 = lax.rsqrt(deg5)

    W1k = jnp.stack([W1A, W1B, W1C, W1D])
    b1k = jnp.stack([b1A, b1B, b1C, b1D])[:, None]
    dinv4k = dinv5.T[:4, :, None]
    y1k = jnp.einsum('nd,kdw->knw', x, W1k) * dinv4k
    acc1 = _sc_l1_scatter()(y1k.reshape(4 * n, D), srcs_l1, dsts_all, ewt,
                            zeros128)
    gcn1k = jax.nn.relu((acc1[:, :n] + y1k) * dinv4k + b1k)

    dinv_o = dinv5[:, 4:5]
    y2h = jnp.einsum('knd,kdhw->hnw', gcn1k, W2.reshape(4, D, 2, D)) \
        * dinv_o[None]
    acc2 = _make_sc_scatter(160)(y2h.reshape(2 * n, D), srcs_all, dsts_all, zeros128)
    b2h = b2.reshape(2, 1, D)
    gcn2h = jax.nn.relu((acc2[:, :n] + y2h) * dinv_o[None] + b2h)

    y3 = jnp.einsum('hnd,hdw->nw', gcn2h, W3.reshape(2, D, D)) * dinv_o
    acc3 = _make_sc_scatter(80)(y3, srcs_half, dsts_half, zeros128)
    gcn3 = (acc3[0, :n] + acc3[1, :n] + y3) * dinv_o + b3

    def _piece_dot(wvec):
        w1 = wvec[:4 * D].reshape(4, D)
        w2 = wvec[4 * D:6 * D].reshape(2, D)
        return (jnp.einsum('knd,kd->n', gcn1k, w1)
                + jnp.einsum('hnd,hd->n', gcn2h, w2)
                + gcn3 @ wvec[6 * D:])

    s_rel = _piece_dot(Wp_rel[:, 0])
    s_root = _piece_dot(Wp_root[:, 0])
    srel_rows = jnp.broadcast_to(s_rel[:, None], (n, 128))
    acc_s = _make_sc_scatter(80)(srel_rows, srcs_half, dsts_half, zeros128)
    agg_s = acc_s[0, :n, 0] + acc_s[1, :n, 0]
    pres = agg_s + bp_rel[0] + s_root

    pieces = (gcn1k[0], gcn1k[1], gcn1k[2], gcn1k[3], gcn2h[0], gcn2h[1], gcn3)
    return _pool_readout(pres, batch, pieces, Wm1, bm1, Wm2, bm2, Wm3, bm3)
